# baseline (speedup 1.0000x reference)
import functools

import jax
import jax.numpy as jnp
from jax import lax
from jax.experimental import pallas as pl
from jax.experimental.pallas import tpu as pltpu
from jax.experimental.pallas import tpu_sc as plsc

_NUM_FIELDS = 26
_VOCAB = 100000
_EMBED_DIM = 32
_PROJ_DIM = 128
_IN_DIM = _NUM_FIELDS * _EMBED_DIM

_OUT_CHUNK = 4096
_MLP_BM = 1024

_INV_SQRT2 = 0.7071067811865476


def _sc_gather_transposed(tables_t, idx_t, batch):
    mesh = plsc.VectorSubcoreMesh(core_axis_name="c", subcore_axis_name="s")
    num_workers = mesh.num_cores * mesh.num_subcores
    num_jobs = _NUM_FIELDS * _EMBED_DIM
    jobs_per_w = num_jobs // num_workers
    n_chunks = batch // _OUT_CHUNK
    vecs_per_chunk = _OUT_CHUNK // 16

    @functools.partial(
        pl.kernel,
        out_type=jax.ShapeDtypeStruct((num_jobs, batch), jnp.float32),
        mesh=mesh,
        compiler_params=pltpu.CompilerParams(needs_layout_passes=False),
        scratch_types=[
            pltpu.VMEM((_VOCAB,), jnp.float32),
            pltpu.VMEM((batch,), jnp.int32),
            pltpu.VMEM((2, _OUT_CHUNK), jnp.float32),
            pltpu.SemaphoreType.DMA,
            pltpu.SemaphoreType.DMA,
            pltpu.SemaphoreType.DMA,
            pltpu.SemaphoreType.DMA,
        ],
    )
    def gather_kernel(tab_hbm, idx_hbm, out_hbm, row_v, idx_v, out_v,
                      rsem, gsem, osem0, osem1):
        wid = lax.axis_index("s") * mesh.num_cores + lax.axis_index("c")
        job0 = wid * jobs_per_w
        osems = (osem0, osem1)

        @pl.loop(0, jobs_per_w)
        def _(t):
            j = job0 + t
            f = j // _EMBED_DIM
            e = j % _EMBED_DIM

            @pl.when(jnp.logical_or(t == 0, e == 0))
            def _():
                pltpu.sync_copy(idx_hbm.at[f], idx_v)

            pltpu.async_copy(tab_hbm.at[f, e], row_v, rsem).wait()

            for cc in range(n_chunks):
                buf = cc % 2
                base = cc * _OUT_CHUNK
                gc = t * n_chunks + cc

                @pl.when(gc >= 2)
                def _():
                    pltpu.make_async_copy(
                        out_v.at[buf],
                        out_hbm.at[j, pl.ds(base, _OUT_CHUNK)],
                        osems[buf],
                    ).wait()

                @plsc.parallel_loop(0, vecs_per_chunk, unroll=8)
                def _(k):
                    pos = k * 16
                    vec_idx = idx_v[pl.ds(base + pos, 16)]
                    out_v[buf, pl.ds(pos, 16)] = plsc.load_gather(
                        row_v, [vec_idx]
                    )

                pltpu.async_copy(
                    out_v.at[buf],
                    out_hbm.at[j, pl.ds(base, _OUT_CHUNK)],
                    osems[buf],
                )

        for buf in range(2):
            pltpu.make_async_copy(
                out_v.at[buf],
                out_hbm.at[job0, pl.ds(buf * _OUT_CHUNK, _OUT_CHUNK)],
                osems[buf],
            ).wait()

    return gather_kernel(tables_t, idx_t)


def _mlp_body(xt_ref, w1_ref, b1_ref, w2_ref, b2_ref, o_ref):
    h = lax.dot_general(
        xt_ref[...], w1_ref[...],
        dimension_numbers=(((0,), (0,)), ((), ())),
        preferred_element_type=jnp.float32,
        precision=lax.Precision.HIGHEST,
    ) + b1_ref[...]
    h = 0.5 * h * (1.0 + lax.erf(h * _INV_SQRT2))
    o_ref[...] = jnp.dot(h, w2_ref[...],
                         preferred_element_type=jnp.float32,
                         precision=lax.Precision.HIGHEST) + b2_ref[...]


def _tc_mlp(xt, W1, b1, W2, b2, batch):
    return pl.pallas_call(
        _mlp_body,
        grid=(batch // _MLP_BM,),
        in_specs=[
            pl.BlockSpec((_IN_DIM, _MLP_BM), lambda i: (0, i)),
            pl.BlockSpec((_IN_DIM, _PROJ_DIM), lambda i: (0, 0)),
            pl.BlockSpec((1, _PROJ_DIM), lambda i: (0, 0)),
            pl.BlockSpec((_PROJ_DIM, _PROJ_DIM), lambda i: (0, 0)),
            pl.BlockSpec((1, _PROJ_DIM), lambda i: (0, 0)),
        ],
        out_specs=pl.BlockSpec((_MLP_BM, _PROJ_DIM), lambda i: (i, 0)),
        out_shape=jax.ShapeDtypeStruct((batch, _PROJ_DIM), jnp.float32),
    )(xt, W1, b1.reshape(1, _PROJ_DIM), W2, b2.reshape(1, _PROJ_DIM))


def kernel(tables, W1, b1, W2, b2, x_cat):
    batch = x_cat.shape[0]
    tables_t = tables.transpose(0, 2, 1)
    idx_t = x_cat.astype(jnp.int32).T
    xt = _sc_gather_transposed(tables_t, idx_t, batch)
    return _tc_mlp(xt, W1, b1, W2, b2, batch)


# speedup vs baseline: 1.1973x; 1.1973x over previous
import functools

import jax
import jax.numpy as jnp
from jax import lax
from jax.experimental import pallas as pl
from jax.experimental.pallas import tpu as pltpu
from jax.experimental.pallas import tpu_sc as plsc

_NUM_FIELDS = 26
_VOCAB = 100000
_EMBED_DIM = 32
_PROJ_DIM = 128
_IN_DIM = _NUM_FIELDS * _EMBED_DIM

_OUT_CHUNK = 4096
_MLP_BM = 1024

_INV_SQRT2 = 0.7071067811865476


def _sc_gather_transposed(tables_t, idx_t, batch):
    mesh = plsc.VectorSubcoreMesh(core_axis_name="c", subcore_axis_name="s")
    num_workers = mesh.num_cores * mesh.num_subcores
    num_jobs = _NUM_FIELDS * _EMBED_DIM
    jobs_per_w = num_jobs // num_workers
    n_chunks = batch // _OUT_CHUNK
    vecs_per_chunk = _OUT_CHUNK // 16

    @functools.partial(
        pl.kernel,
        out_type=jax.ShapeDtypeStruct((num_jobs, batch), jnp.float32),
        mesh=mesh,
        compiler_params=pltpu.CompilerParams(needs_layout_passes=False),
        scratch_types=[
            pltpu.VMEM((_VOCAB,), jnp.float32),
            pltpu.VMEM((batch,), jnp.int32),
            pltpu.VMEM((2, _OUT_CHUNK), jnp.float32),
            pltpu.SemaphoreType.DMA,
            pltpu.SemaphoreType.DMA,
            pltpu.SemaphoreType.DMA,
            pltpu.SemaphoreType.DMA,
        ],
    )
    def gather_kernel(tab_hbm, idx_hbm, out_hbm, row_v, idx_v, out_v,
                      rsem, gsem, osem0, osem1):
        wid = lax.axis_index("s") * mesh.num_cores + lax.axis_index("c")
        job0 = wid * jobs_per_w
        osems = (osem0, osem1)

        @pl.loop(0, jobs_per_w)
        def _(t):
            j = job0 + t
            f = j // _EMBED_DIM
            e = j % _EMBED_DIM

            @pl.when(jnp.logical_or(t == 0, e == 0))
            def _():
                pltpu.sync_copy(idx_hbm.at[f], idx_v)

            pltpu.async_copy(tab_hbm.at[f, e], row_v, rsem).wait()

            for cc in range(n_chunks):
                buf = cc % 2
                base = cc * _OUT_CHUNK
                gc = t * n_chunks + cc

                @pl.when(gc >= 2)
                def _():
                    pltpu.make_async_copy(
                        out_v.at[buf],
                        out_hbm.at[j, pl.ds(base, _OUT_CHUNK)],
                        osems[buf],
                    ).wait()

                @plsc.parallel_loop(0, vecs_per_chunk, unroll=8)
                def _(k):
                    pos = k * 16
                    vec_idx = idx_v[pl.ds(base + pos, 16)]
                    out_v[buf, pl.ds(pos, 16)] = plsc.load_gather(
                        row_v, [vec_idx]
                    )

                pltpu.async_copy(
                    out_v.at[buf],
                    out_hbm.at[j, pl.ds(base, _OUT_CHUNK)],
                    osems[buf],
                )

        for buf in range(2):
            pltpu.make_async_copy(
                out_v.at[buf],
                out_hbm.at[job0, pl.ds(buf * _OUT_CHUNK, _OUT_CHUNK)],
                osems[buf],
            ).wait()

    return gather_kernel(tables_t, idx_t)


def _mlp_body(xt_ref, w1_ref, b1_ref, w2_ref, b2_ref, o_ref):
    h = lax.dot_general(
        xt_ref[...], w1_ref[...],
        dimension_numbers=(((0,), (0,)), ((), ())),
        preferred_element_type=jnp.float32,
        precision=lax.Precision.DEFAULT,
    ) + b1_ref[...]
    h = 0.5 * h * (1.0 + lax.erf(h * _INV_SQRT2))
    o_ref[...] = jnp.dot(h, w2_ref[...],
                         preferred_element_type=jnp.float32,
                         precision=lax.Precision.DEFAULT) + b2_ref[...]


def _tc_mlp(xt, W1, b1, W2, b2, batch):
    return pl.pallas_call(
        _mlp_body,
        grid=(batch // _MLP_BM,),
        in_specs=[
            pl.BlockSpec((_IN_DIM, _MLP_BM), lambda i: (0, i)),
            pl.BlockSpec((_IN_DIM, _PROJ_DIM), lambda i: (0, 0)),
            pl.BlockSpec((1, _PROJ_DIM), lambda i: (0, 0)),
            pl.BlockSpec((_PROJ_DIM, _PROJ_DIM), lambda i: (0, 0)),
            pl.BlockSpec((1, _PROJ_DIM), lambda i: (0, 0)),
        ],
        out_specs=pl.BlockSpec((_MLP_BM, _PROJ_DIM), lambda i: (i, 0)),
        out_shape=jax.ShapeDtypeStruct((batch, _PROJ_DIM), jnp.float32),
    )(xt, W1, b1.reshape(1, _PROJ_DIM), W2, b2.reshape(1, _PROJ_DIM))


def kernel(tables, W1, b1, W2, b2, x_cat):
    batch = x_cat.shape[0]
    tables_t = tables.transpose(0, 2, 1)
    idx_t = x_cat.astype(jnp.int32).T
    xt = _sc_gather_transposed(tables_t, idx_t, batch)
    return _tc_mlp(xt, W1, b1, W2, b2, batch)


# speedup vs baseline: 1.2000x; 1.0023x over previous
import functools

import jax
import jax.numpy as jnp
from jax import lax
from jax.experimental import pallas as pl
from jax.experimental.pallas import tpu as pltpu
from jax.experimental.pallas import tpu_sc as plsc

_NUM_FIELDS = 26
_VOCAB = 100000
_EMBED_DIM = 32
_PROJ_DIM = 128
_IN_DIM = _NUM_FIELDS * _EMBED_DIM

_OUT_CHUNK = 4096
_MLP_BM = 1024

_INV_SQRT2 = 0.7071067811865476


def _sc_gather_transposed(tables_t, idx_t, batch):
    mesh = plsc.VectorSubcoreMesh(core_axis_name="c", subcore_axis_name="s")
    num_workers = mesh.num_cores * mesh.num_subcores
    num_jobs = _NUM_FIELDS * _EMBED_DIM
    jobs_per_w = num_jobs // num_workers
    n_chunks = batch // _OUT_CHUNK
    vecs_per_chunk = _OUT_CHUNK // 16

    @functools.partial(
        pl.kernel,
        out_type=jax.ShapeDtypeStruct((num_jobs, batch), jnp.float32),
        mesh=mesh,
        compiler_params=pltpu.CompilerParams(needs_layout_passes=False),
        scratch_types=[
            pltpu.VMEM((_VOCAB,), jnp.float32),
            pltpu.VMEM((batch,), jnp.int32),
            pltpu.VMEM((2, _OUT_CHUNK), jnp.float32),
            pltpu.SemaphoreType.DMA,
            pltpu.SemaphoreType.DMA,
            pltpu.SemaphoreType.DMA,
            pltpu.SemaphoreType.DMA,
        ],
    )
    def gather_kernel(tab_hbm, idx_hbm, out_hbm, row_v, idx_v, out_v,
                      rsem, gsem, osem0, osem1):
        wid = lax.axis_index("s") * mesh.num_cores + lax.axis_index("c")
        job0 = wid * jobs_per_w
        osems = (osem0, osem1)

        @pl.loop(0, jobs_per_w)
        def _(t):
            j = job0 + t
            f = j // _EMBED_DIM
            e = j % _EMBED_DIM

            @pl.when(jnp.logical_or(t == 0, e == 0))
            def _():
                pltpu.sync_copy(idx_hbm.at[f], idx_v)

            pltpu.async_copy(tab_hbm.at[f, e], row_v, rsem).wait()

            for cc in range(n_chunks):
                buf = cc % 2
                base = cc * _OUT_CHUNK
                gc = t * n_chunks + cc

                @pl.when(gc >= 2)
                def _():
                    pltpu.make_async_copy(
                        out_v.at[buf],
                        out_hbm.at[j, pl.ds(base, _OUT_CHUNK)],
                        osems[buf],
                    ).wait()

                @plsc.parallel_loop(0, vecs_per_chunk, unroll=16)
                def _(k):
                    pos = k * 16
                    vec_idx = idx_v[pl.ds(base + pos, 16)]
                    out_v[buf, pl.ds(pos, 16)] = plsc.load_gather(
                        row_v, [vec_idx]
                    )

                pltpu.async_copy(
                    out_v.at[buf],
                    out_hbm.at[j, pl.ds(base, _OUT_CHUNK)],
                    osems[buf],
                )

        for buf in range(2):
            pltpu.make_async_copy(
                out_v.at[buf],
                out_hbm.at[job0, pl.ds(buf * _OUT_CHUNK, _OUT_CHUNK)],
                osems[buf],
            ).wait()

    return gather_kernel(tables_t, idx_t)


def _mlp_body(xt_ref, w1_ref, b1_ref, w2_ref, b2_ref, o_ref):
    h = lax.dot_general(
        xt_ref[...], w1_ref[...],
        dimension_numbers=(((0,), (0,)), ((), ())),
        preferred_element_type=jnp.float32,
        precision=lax.Precision.DEFAULT,
    ) + b1_ref[...]
    h = 0.5 * h * (1.0 + lax.erf(h * _INV_SQRT2))
    o_ref[...] = jnp.dot(h, w2_ref[...],
                         preferred_element_type=jnp.float32,
                         precision=lax.Precision.DEFAULT) + b2_ref[...]


def _tc_mlp(xt, W1, b1, W2, b2, batch):
    return pl.pallas_call(
        _mlp_body,
        grid=(batch // _MLP_BM,),
        in_specs=[
            pl.BlockSpec((_IN_DIM, _MLP_BM), lambda i: (0, i)),
            pl.BlockSpec((_IN_DIM, _PROJ_DIM), lambda i: (0, 0)),
            pl.BlockSpec((1, _PROJ_DIM), lambda i: (0, 0)),
            pl.BlockSpec((_PROJ_DIM, _PROJ_DIM), lambda i: (0, 0)),
            pl.BlockSpec((1, _PROJ_DIM), lambda i: (0, 0)),
        ],
        out_specs=pl.BlockSpec((_MLP_BM, _PROJ_DIM), lambda i: (i, 0)),
        out_shape=jax.ShapeDtypeStruct((batch, _PROJ_DIM), jnp.float32),
        compiler_params=pltpu.CompilerParams(
            dimension_semantics=("parallel",)),
    )(xt, W1, b1.reshape(1, _PROJ_DIM), W2, b2.reshape(1, _PROJ_DIM))


def kernel(tables, W1, b1, W2, b2, x_cat):
    batch = x_cat.shape[0]
    tables_t = tables.transpose(0, 2, 1)
    idx_t = x_cat.astype(jnp.int32).T
    xt = _sc_gather_transposed(tables_t, idx_t, batch)
    return _tc_mlp(xt, W1, b1, W2, b2, batch)
